# baseline (speedup 1.0000x reference)
import functools

import jax
import jax.numpy as jnp
from jax import lax
from jax.experimental import pallas as pl
from jax.experimental.pallas import tpu as pltpu
from jax.experimental.pallas import tpu_sc as plsc

D = 16


def _make_sc_kernel(batch: int):
    info = plsc.get_sparse_core_info()
    nc, ns, nl = info.num_cores, info.num_subcores, info.num_lanes
    nw = nc * ns
    assert nl == D
    assert batch % (8 * nw) == 0
    chunk = batch // nw

    mesh = plsc.VectorSubcoreMesh(core_axis_name="c", subcore_axis_name="s")

    @functools.partial(
        pl.kernel,
        mesh=mesh,
        out_type=jax.ShapeDtypeStruct((batch,), jnp.float32),
        scratch_types=[
            pltpu.VMEM((chunk,), jnp.int32),
            pltpu.VMEM((chunk,), jnp.int32),
            pltpu.VMEM((D, chunk), jnp.float32),
            pltpu.VMEM((D, chunk), jnp.float32),
            pltpu.VMEM((chunk,), jnp.float32),
            pltpu.SemaphoreType.DMA,
            pltpu.SemaphoreType.DMA,
        ],
        compiler_params=pltpu.CompilerParams(use_tc_tiling_on_sc=False),
    )
    def sc_kernel(uid_idx_hbm, iid_idx_hbm, ut_hbm, vt_hbm,
                  out_hbm, idx_u, idx_i, u_bufs, v_bufs, out_v, sem_u, sem_i):
        wid = lax.axis_index("s") * nc + lax.axis_index("c")
        base = wid * chunk

        pltpu.sync_copy(uid_idx_hbm.at[pl.ds(base, chunk)], idx_u)
        pltpu.sync_copy(iid_idx_hbm.at[pl.ds(base, chunk)], idx_i)

        copies = []
        for d in range(D):
            copies.append(
                pltpu.async_copy(ut_hbm.at[d].at[idx_u], u_bufs.at[d], sem_u))
            copies.append(
                pltpu.async_copy(vt_hbm.at[d].at[idx_i], v_bufs.at[d], sem_i))
        for c in copies:
            c.wait()

        def compute(g, carry):
            blk = pl.ds(g * D, D)
            acc = u_bufs[0, blk] * v_bufs[0, blk]
            for d in range(1, D):
                acc = acc + u_bufs[d, blk] * v_bufs[d, blk]
            out_v[blk] = acc
            return carry

        lax.fori_loop(0, chunk // D, compute, 0)

        pltpu.sync_copy(out_v, out_hbm.at[pl.ds(base, chunk)])

    return sc_kernel


def kernel(x, uid_table, iid_table):
    batch = x.shape[0]
    xi = x.astype(jnp.int32)
    uid_idx = xi[:, 0]
    iid_idx = xi[:, 1]
    sc = _make_sc_kernel(batch)
    return sc(uid_idx, iid_idx, uid_table.T, iid_table.T)


# speedup vs baseline: 14.6601x; 14.6601x over previous
import functools

import jax
import jax.numpy as jnp
from jax import lax
from jax.experimental import pallas as pl
from jax.experimental.pallas import tpu as pltpu
from jax.experimental.pallas import tpu_sc as plsc

D = 16
NB = 7813
B2 = 256
GJ = -(-NB // B2)
FLAT = 2 * NB * 8 * 128
HALF = NB * 8 * 128


def _detile(ut):
    def body(in_ref, out_ref):
        x = in_ref[...].reshape(8, B2, 128)
        out_ref[...] = jnp.transpose(x, (1, 0, 2)).reshape(1, B2, 8, 128)

    return pl.pallas_call(
        body,
        grid=(2, GJ),
        in_specs=[pl.BlockSpec((8, B2 * 128), lambda i, j: (i, j))],
        out_specs=pl.BlockSpec((1, B2, 8, 128), lambda i, j: (i, j, 0, 0)),
        out_shape=jax.ShapeDtypeStruct((2, NB, 8, 128), jnp.float32),
    )(ut)


def _make_sc_kernel(batch: int):
    info = plsc.get_sparse_core_info()
    nc, ns, nl = info.num_cores, info.num_subcores, info.num_lanes
    nw = nc * ns
    assert nl == D
    assert batch % (8 * nw) == 0
    chunk = batch // nw

    mesh = plsc.VectorSubcoreMesh(core_axis_name="c", subcore_axis_name="s")

    @functools.partial(
        pl.kernel,
        mesh=mesh,
        out_type=jax.ShapeDtypeStruct((batch,), jnp.float32),
        scratch_types=[
            pltpu.VMEM((chunk,), jnp.int32),
            pltpu.VMEM((chunk,), jnp.int32),
            pltpu.VMEM((D, chunk), jnp.float32),
            pltpu.VMEM((D, chunk), jnp.float32),
            pltpu.VMEM((chunk,), jnp.float32),
            pltpu.SemaphoreType.DMA,
            pltpu.SemaphoreType.DMA,
        ],
        compiler_params=pltpu.CompilerParams(use_tc_tiling_on_sc=False),
    )
    def sc_kernel(uid_idx_hbm, iid_idx_hbm, lin_u_hbm, lin_v_hbm,
                  out_hbm, idx_u, idx_i, u_bufs, v_bufs, out_v, sem_u, sem_i):
        wid = lax.axis_index("s") * nc + lax.axis_index("c")
        base = wid * chunk

        pltpu.sync_copy(uid_idx_hbm.at[pl.ds(base, chunk)], idx_u)
        pltpu.sync_copy(iid_idx_hbm.at[pl.ds(base, chunk)], idx_i)

        def to_base(g, carry):
            blk = pl.ds(g * D, D)
            rv = idx_u[blk]
            idx_u[blk] = ((rv >> 7) << 10) | (rv & 127)
            sv = idx_i[blk]
            idx_i[blk] = ((sv >> 7) << 10) | (sv & 127)
            return carry

        lax.fori_loop(0, chunk // D, to_base, 0)

        copies = []
        for d in range(D):
            c_d = (d // 8) * HALF + (d % 8) * 128
            l_d = FLAT - c_d
            copies.append(pltpu.async_copy(
                lin_u_hbm.at[pl.ds(c_d, l_d)].at[idx_u], u_bufs.at[d], sem_u))
            copies.append(pltpu.async_copy(
                lin_v_hbm.at[pl.ds(c_d, l_d)].at[idx_i], v_bufs.at[d], sem_i))
        for c in copies:
            c.wait()

        def compute(g, carry):
            blk = pl.ds(g * D, D)
            acc = u_bufs[0, blk] * v_bufs[0, blk]
            for d in range(1, D):
                acc = acc + u_bufs[d, blk] * v_bufs[d, blk]
            out_v[blk] = acc
            return carry

        lax.fori_loop(0, chunk // D, compute, 0)

        pltpu.sync_copy(out_v, out_hbm.at[pl.ds(base, chunk)])

    return sc_kernel


def kernel(x, uid_table, iid_table):
    batch = x.shape[0]
    xi = x.astype(jnp.int32)
    uid_idx = xi[:, 0]
    iid_idx = xi[:, 1]
    lin_u = _detile(uid_table.T).reshape(FLAT)
    lin_v = _detile(iid_table.T).reshape(FLAT)
    sc = _make_sc_kernel(batch)
    return sc(uid_idx, iid_idx, lin_u, lin_v)


# speedup vs baseline: 21.2435x; 1.4491x over previous
import functools

import jax
import jax.numpy as jnp
from jax import lax
from jax.experimental import pallas as pl
from jax.experimental.pallas import tpu as pltpu
from jax.experimental.pallas import tpu_sc as plsc

D = 16
NB = 7813
B2 = 1024
GJ = -(-NB // B2)
FLAT = 2 * NB * 8 * 128
HALF = NB * 8 * 128

_SC_PARAMS = pltpu.CompilerParams(use_tc_tiling_on_sc=False)


def _detile(t):
    def body(in_ref, out_ref):
        x = in_ref[...].reshape(8, B2, 128)
        out_ref[...] = jnp.transpose(x, (1, 0, 2)).reshape(1, B2, 8, 128)

    return pl.pallas_call(
        body,
        grid=(2, GJ),
        in_specs=[pl.BlockSpec((8, B2 * 128), lambda i, j: (i, j))],
        out_specs=pl.BlockSpec((1, B2, 8, 128), lambda i, j: (i, j, 0, 0)),
        out_shape=jax.ShapeDtypeStruct((2, NB, 8, 128), jnp.float32),
    )(t)


def _to_base(idx_ref, chunk):
    def body(g, carry):
        blk = pl.ds(g * D, D)
        rv = idx_ref[blk]
        idx_ref[blk] = ((rv >> 7) << 10) | (rv & 127)
        return carry

    lax.fori_loop(0, chunk // D, body, 0)


def _fire_gathers(lin_hbm, idx_ref, bufs, sem):
    copies = []
    for d in range(D):
        c_d = (d // 8) * HALF + (d % 8) * 128
        copies.append(pltpu.async_copy(
            lin_hbm.at[pl.ds(c_d, FLAT - c_d)].at[idx_ref], bufs.at[d], sem))
    return copies


def _make_sc_a(batch, nc, nw, chunk, mesh):
    @functools.partial(
        pl.kernel,
        mesh=mesh,
        out_type=jax.ShapeDtypeStruct((nw, D, chunk), jnp.float32),
        scratch_types=[
            pltpu.VMEM((chunk,), jnp.int32),
            pltpu.VMEM((D, chunk), jnp.float32),
            pltpu.SemaphoreType.DMA,
        ],
        compiler_params=_SC_PARAMS,
    )
    def sc_a(uid_idx_hbm, lin_u_hbm, out_hbm, idx_u, u_bufs, sem_u):
        wid = lax.axis_index("s") * nc + lax.axis_index("c")
        base = wid * chunk
        pltpu.sync_copy(uid_idx_hbm.at[pl.ds(base, chunk)], idx_u)
        _to_base(idx_u, chunk)
        for c in _fire_gathers(lin_u_hbm, idx_u, u_bufs, sem_u):
            c.wait()
        pltpu.sync_copy(u_bufs, out_hbm.at[wid])

    return sc_a


def _make_sc_b(batch, nc, nw, chunk, mesh):
    @functools.partial(
        pl.kernel,
        mesh=mesh,
        out_type=jax.ShapeDtypeStruct((batch,), jnp.float32),
        scratch_types=[
            pltpu.VMEM((chunk,), jnp.int32),
            pltpu.VMEM((D, chunk), jnp.float32),
            pltpu.VMEM((D, chunk), jnp.float32),
            pltpu.VMEM((chunk,), jnp.float32),
            pltpu.SemaphoreType.DMA,
            pltpu.SemaphoreType.DMA,
        ],
        compiler_params=_SC_PARAMS,
    )
    def sc_b(iid_idx_hbm, lin_v_hbm, urows_hbm, out_hbm,
             idx_i, u_bufs, v_bufs, out_v, sem_u, sem_i):
        wid = lax.axis_index("s") * nc + lax.axis_index("c")
        base = wid * chunk
        pltpu.sync_copy(iid_idx_hbm.at[pl.ds(base, chunk)], idx_i)
        _to_base(idx_i, chunk)
        copies = _fire_gathers(lin_v_hbm, idx_i, v_bufs, sem_i)
        cu = pltpu.async_copy(urows_hbm.at[wid], u_bufs, sem_u)
        for c in copies:
            c.wait()
        cu.wait()

        def compute(g, carry):
            blk = pl.ds(g * D, D)
            acc = u_bufs[0, blk] * v_bufs[0, blk]
            for d in range(1, D):
                acc = acc + u_bufs[d, blk] * v_bufs[d, blk]
            out_v[blk] = acc
            return carry

        lax.fori_loop(0, chunk // D, compute, 0)
        pltpu.sync_copy(out_v, out_hbm.at[pl.ds(base, chunk)])

    return sc_b


def kernel(x, uid_table, iid_table):
    batch = x.shape[0]
    xi = x.astype(jnp.int32)
    uid_idx = xi[:, 0]
    iid_idx = xi[:, 1]

    info = plsc.get_sparse_core_info()
    nc, ns, nl = info.num_cores, info.num_subcores, info.num_lanes
    nw = nc * ns
    assert nl == D and batch % (8 * nw) == 0
    chunk = batch // nw
    mesh = plsc.VectorSubcoreMesh(core_axis_name="c", subcore_axis_name="s")

    lin_u = _detile(uid_table.T).reshape(FLAT)
    u_rows = _make_sc_a(batch, nc, nw, chunk, mesh)(uid_idx, lin_u)
    lin_v = _detile(iid_table.T).reshape(FLAT)
    return _make_sc_b(batch, nc, nw, chunk, mesh)(iid_idx, lin_v, u_rows)


# speedup vs baseline: 21.5996x; 1.0168x over previous
import functools

import jax
import jax.numpy as jnp
from jax import lax
from jax.experimental import pallas as pl
from jax.experimental.pallas import tpu as pltpu
from jax.experimental.pallas import tpu_sc as plsc

D = 16
NB = 7813
B2 = 1024
GJ = -(-NB // B2)
FLAT = 2 * NB * 8 * 128
HALF = NB * 8 * 128


def _detile(ut, vt):
    def body(u_ref, v_ref, ou_ref, ov_ref):
        xu = u_ref[...].reshape(8, B2, 128)
        ou_ref[...] = jnp.transpose(xu, (1, 0, 2)).reshape(1, B2, 8, 128)
        xv = v_ref[...].reshape(8, B2, 128)
        ov_ref[...] = jnp.transpose(xv, (1, 0, 2)).reshape(1, B2, 8, 128)

    spec_in = pl.BlockSpec((8, B2 * 128), lambda i, j: (i, j))
    spec_out = pl.BlockSpec((1, B2, 8, 128), lambda i, j: (i, j, 0, 0))
    shape_out = jax.ShapeDtypeStruct((2, NB, 8, 128), jnp.float32)
    return pl.pallas_call(
        body,
        grid=(2, GJ),
        in_specs=[spec_in, spec_in],
        out_specs=[spec_out, spec_out],
        out_shape=[shape_out, shape_out],
    )(ut, vt)


def _make_sc_kernel(batch: int):
    info = plsc.get_sparse_core_info()
    nc, ns, nl = info.num_cores, info.num_subcores, info.num_lanes
    nw = nc * ns
    assert nl == D
    assert batch % (8 * nw) == 0
    chunk = batch // nw

    mesh = plsc.VectorSubcoreMesh(core_axis_name="c", subcore_axis_name="s")

    @functools.partial(
        pl.kernel,
        mesh=mesh,
        out_type=jax.ShapeDtypeStruct((batch,), jnp.float32),
        scratch_types=[
            pltpu.VMEM((chunk,), jnp.int32),
            pltpu.VMEM((chunk,), jnp.int32),
            pltpu.VMEM((D, chunk), jnp.float32),
            pltpu.VMEM((D, chunk), jnp.float32),
            pltpu.VMEM((chunk,), jnp.float32),
            pltpu.SemaphoreType.DMA,
            pltpu.SemaphoreType.DMA,
        ],
        compiler_params=pltpu.CompilerParams(use_tc_tiling_on_sc=False),
    )
    def sc_kernel(uid_idx_hbm, iid_idx_hbm, lin_u_hbm, lin_v_hbm,
                  out_hbm, idx_u, idx_i, u_bufs, v_bufs, out_v, sem_u, sem_i):
        wid = lax.axis_index("s") * nc + lax.axis_index("c")
        base = wid * chunk

        pltpu.sync_copy(uid_idx_hbm.at[pl.ds(base, chunk)], idx_u)
        pltpu.sync_copy(iid_idx_hbm.at[pl.ds(base, chunk)], idx_i)

        def to_base(g, carry):
            blk = pl.ds(g * D, D)
            rv = idx_u[blk]
            idx_u[blk] = ((rv >> 7) << 10) | (rv & 127)
            sv = idx_i[blk]
            idx_i[blk] = ((sv >> 7) << 10) | (sv & 127)
            return carry

        lax.fori_loop(0, chunk // D, to_base, 0)

        copies = []
        for d in range(D):
            c_d = (d // 8) * HALF + (d % 8) * 128
            l_d = FLAT - c_d
            copies.append(pltpu.async_copy(
                lin_u_hbm.at[pl.ds(c_d, l_d)].at[idx_u], u_bufs.at[d], sem_u))
            copies.append(pltpu.async_copy(
                lin_v_hbm.at[pl.ds(c_d, l_d)].at[idx_i], v_bufs.at[d], sem_i))
        for c in copies:
            c.wait()

        def compute(g, carry):
            blk = pl.ds(g * D, D)
            acc = u_bufs[0, blk] * v_bufs[0, blk]
            for d in range(1, D):
                acc = acc + u_bufs[d, blk] * v_bufs[d, blk]
            out_v[blk] = acc
            return carry

        lax.fori_loop(0, chunk // D, compute, 0)

        pltpu.sync_copy(out_v, out_hbm.at[pl.ds(base, chunk)])

    return sc_kernel


def kernel(x, uid_table, iid_table):
    batch = x.shape[0]
    xi = x.astype(jnp.int32)
    uid_idx = xi[:, 0]
    iid_idx = xi[:, 1]
    lu4, lv4 = _detile(uid_table.T, iid_table.T)
    lin_u = lu4.reshape(FLAT)
    lin_v = lv4.reshape(FLAT)
    sc = _make_sc_kernel(batch)
    return sc(uid_idx, iid_idx, lin_u, lin_v)
